# baseline (speedup 1.0000x reference)
import functools

import jax
import jax.numpy as jnp
from jax import lax
from jax.experimental import pallas as pl
from jax.experimental.pallas import tpu as pltpu
from jax.experimental.pallas import tpu_sc as plsc

NC = 2
NS = 16
NW = NC * NS
L = 16

B = 16384
D = 128
ROWS_PER_W = B // NW
CHUNK = 128
NCHUNK = ROWS_PER_W // CHUNK
VECS = CHUNK * (D // L)


def _center_loss_body(feat_hbm, idx_hbm, centers_hbm, out_hbm,
                      idx_v, rows_v, feats_v, acc_v, sem):
    wid = lax.axis_index("s") * NC + lax.axis_index("c")
    base = wid * ROWS_PER_W

    pltpu.sync_copy(idx_hbm.at[wid], idx_v)

    acc = jnp.zeros((L,), jnp.float32)

    def vec_body(k, a):
        i = k >> 3
        j = (k & 7) * L
        f = feats_v[i, pl.ds(j, L)]
        c = rows_v[i, pl.ds(j, L)]
        d = f - c
        return a + d * d

    for ci in range(NCHUNK):
        pltpu.async_copy(centers_hbm.at[idx_v.at[ci]], rows_v, sem).wait()
        pltpu.sync_copy(feat_hbm.at[pl.ds(base + ci * CHUNK, CHUNK)], feats_v)
        acc = lax.fori_loop(0, VECS, vec_body, acc)

    acc_v[...] = acc
    pltpu.sync_copy(acc_v, out_hbm.at[wid])


@jax.jit
def kernel(features, labels, centers):
    idx = labels.astype(jnp.int32).reshape(NW, NCHUNK, CHUNK)
    call = pl.kernel(
        _center_loss_body,
        out_type=jax.ShapeDtypeStruct((NW, L), jnp.float32),
        mesh=plsc.VectorSubcoreMesh(core_axis_name="c", subcore_axis_name="s"),
        scratch_types=[
            pltpu.VMEM((NCHUNK, CHUNK), jnp.int32),
            pltpu.VMEM((CHUNK, D), jnp.float32),
            pltpu.VMEM((CHUNK, D), jnp.float32),
            pltpu.VMEM((L,), jnp.float32),
            pltpu.SemaphoreType.DMA,
        ],
    )
    partials = call(features, idx, centers)
    return jnp.sum(partials) / B


# speedup vs baseline: 1.5377x; 1.5377x over previous
import functools

import jax
import jax.numpy as jnp
from jax import lax
from jax.experimental import pallas as pl
from jax.experimental.pallas import tpu as pltpu
from jax.experimental.pallas import tpu_sc as plsc

NC = 2
NS = 16
NW = NC * NS
L = 16

B = 16384
D = 128
JG = D // L
ROWS_PER_W = B // NW
CHUNK = 128
NCHUNK = ROWS_PER_W // CHUNK


def _center_loss_body(feat_hbm, idx_hbm, centers_hbm, out_hbm,
                      idx_v, rows_v, feats_v, acc_v,
                      sg0, sg1, sf0, sf1):
    wid = lax.axis_index("s") * NC + lax.axis_index("c")
    base = wid * ROWS_PER_W

    pltpu.sync_copy(idx_hbm.at[wid], idx_v)

    sg = [sg0, sg1]
    sf = [sf0, sf1]
    gd = [None, None]
    fd = [None, None]

    gd[0] = pltpu.async_copy(centers_hbm.at[idx_v.at[0]], rows_v.at[0], sg[0])
    fd[0] = pltpu.async_copy(feat_hbm.at[pl.ds(base, CHUNK)], feats_v.at[0],
                             sf[0])

    accs = tuple(jnp.zeros((L,), jnp.float32) for _ in range(JG))

    for ci in range(NCHUNK):
        b = ci % 2
        nb = (ci + 1) % 2
        if ci + 1 < NCHUNK:
            gd[nb] = pltpu.async_copy(centers_hbm.at[idx_v.at[ci + 1]],
                                      rows_v.at[nb], sg[nb])
            fd[nb] = pltpu.async_copy(
                feat_hbm.at[pl.ds(base + (ci + 1) * CHUNK, CHUNK)],
                feats_v.at[nb], sf[nb])
        gd[b].wait()
        fd[b].wait()

        def row_body(i, accs, b=b):
            out = []
            for j in range(JG):
                f = feats_v[b, i, pl.ds(j * L, L)]
                c = rows_v[b, i, pl.ds(j * L, L)]
                d = f - c
                out.append(accs[j] + d * d)
            return tuple(out)

        accs = lax.fori_loop(0, CHUNK, row_body, accs)

    acc = accs[0]
    for j in range(1, JG):
        acc = acc + accs[j]
    acc_v[...] = acc
    pltpu.sync_copy(acc_v, out_hbm.at[wid])


@jax.jit
def kernel(features, labels, centers):
    idx = labels.astype(jnp.int32).reshape(NW, NCHUNK, CHUNK)
    call = pl.kernel(
        _center_loss_body,
        out_type=jax.ShapeDtypeStruct((NW, L), jnp.float32),
        mesh=plsc.VectorSubcoreMesh(core_axis_name="c", subcore_axis_name="s"),
        scratch_types=[
            pltpu.VMEM((NCHUNK, CHUNK), jnp.int32),
            pltpu.VMEM((2, CHUNK, D), jnp.float32),
            pltpu.VMEM((2, CHUNK, D), jnp.float32),
            pltpu.VMEM((L,), jnp.float32),
            pltpu.SemaphoreType.DMA,
            pltpu.SemaphoreType.DMA,
            pltpu.SemaphoreType.DMA,
            pltpu.SemaphoreType.DMA,
        ],
    )
    partials = call(features, idx, centers)
    return jnp.sum(partials) / B
